# baseline (speedup 1.0000x reference)
import functools

import jax
import jax.numpy as jnp
from jax import lax
from jax.experimental import pallas as pl
from jax.experimental.pallas import tpu as pltpu
from jax.experimental.pallas import tpu_sc as plsc

_NC = 2
_NS = 16
_NW = _NC * _NS
_L = 16

_N = 50000
_EMB = 384
_MEM = 128
_MAX_NODES = 250000
_OUT = 2

_sc_mesh = plsc.VectorSubcoreMesh(
    core_axis_name="c", subcore_axis_name="s", num_cores=_NC, num_subcores=_NS
)


def _wid():
    return lax.axis_index("s") * _NC + lax.axis_index("c")


_BPW = 1568
_B_PAD = _BPW * _NW
_GCH = 392
_NGC = _BPW // _GCH


@functools.partial(
    pl.kernel,
    out_type=jax.ShapeDtypeStruct((_B_PAD, _MEM), jnp.float32),
    mesh=_sc_mesh,
    scratch_types=[
        pltpu.VMEM((_BPW,), jnp.int32),
        pltpu.VMEM((_GCH, _MEM), jnp.float32),
        pltpu.VMEM((_GCH, _MEM), jnp.float32),
        pltpu.SemaphoreType.DMA,
    ],
    cost_estimate=pl.CostEstimate(
        flops=0, transcendentals=0, bytes_accessed=110_000_000),
    name="sc_gather_rows",
)
def _sc_gather(mem_hbm, ids_hbm, out_hbm, idx_v, rows_a, rows_b, sem):
    base = _wid() * _BPW
    pltpu.sync_copy(ids_hbm.at[pl.ds(base, _BPW)], idx_v)
    bufs = [rows_a, rows_b]

    def _idx(k):
        return idx_v.at[pl.ds(k * _GCH, _GCH)]

    pltpu.async_copy(mem_hbm.at[_idx(0)], bufs[0], sem)
    for k in range(_NGC):
        if k + 1 < _NGC:
            pltpu.async_copy(mem_hbm.at[_idx(k + 1)], bufs[(k + 1) % 2], sem)
        pltpu.make_async_copy(mem_hbm.at[_idx(k)], bufs[k % 2], sem).wait()
        pltpu.sync_copy(bufs[k % 2], out_hbm.at[pl.ds(base + k * _GCH, _GCH)])


_RB = 1024


def _tc_body(emb_ref, prev_ref, wih_ref, whh_ref, bih_ref, bhh_ref,
             wce_ref, wcm_ref, bcls_ref, nm_ref, lg_ref):
    emb = emb_ref[...]
    prev = prev_ref[...]
    emb16 = emb.astype(jnp.bfloat16)
    prev16 = prev.astype(jnp.bfloat16)
    gi = jnp.dot(emb16, wih_ref[...], preferred_element_type=jnp.float32) + bih_ref[...]
    gh = jnp.dot(prev16, whh_ref[...], preferred_element_type=jnp.float32) + bhh_ref[...]
    r = jax.nn.sigmoid(gi[:, :_MEM] + gh[:, :_MEM])
    z = jax.nn.sigmoid(gi[:, _MEM:2 * _MEM] + gh[:, _MEM:2 * _MEM])
    n = jnp.tanh(gi[:, 2 * _MEM:] + r * gh[:, 2 * _MEM:])
    nm = (1.0 - z) * n + z * prev
    nm_ref[...] = nm
    dn = (((0,), (1,)), ((), ()))
    lg_ref[...] = (
        lax.dot_general(wce_ref[...], emb16, dn, preferred_element_type=jnp.float32)
        + lax.dot_general(wcm_ref[...], nm.astype(jnp.bfloat16), dn,
                          preferred_element_type=jnp.float32)
        + bcls_ref[...]
    )


_tc_call = pl.pallas_call(
    _tc_body,
    grid=(_B_PAD // _RB,),
    in_specs=[
        pl.BlockSpec((_RB, _EMB), lambda i: (i, 0)),
        pl.BlockSpec((_RB, _MEM), lambda i: (i, 0)),
        pl.BlockSpec((_EMB, 3 * _MEM), lambda i: (0, 0)),
        pl.BlockSpec((_MEM, 3 * _MEM), lambda i: (0, 0)),
        pl.BlockSpec((1, 3 * _MEM), lambda i: (0, 0)),
        pl.BlockSpec((1, 3 * _MEM), lambda i: (0, 0)),
        pl.BlockSpec((_EMB, _OUT), lambda i: (0, 0)),
        pl.BlockSpec((_MEM, _OUT), lambda i: (0, 0)),
        pl.BlockSpec((_OUT, 1), lambda i: (0, 0)),
    ],
    out_specs=[
        pl.BlockSpec((_RB, _MEM), lambda i: (i, 0)),
        pl.BlockSpec((_OUT, _RB), lambda i: (0, i)),
    ],
    out_shape=[
        jax.ShapeDtypeStruct((_B_PAD, _MEM), jnp.float32),
        jax.ShapeDtypeStruct((_OUT, _B_PAD), jnp.float32),
    ],
)


_RANGE = 8192
_SHIFT = 13
_IDC = 10000
_NIDC = _N // _IDC
_CS = 256
_CAP = _RANGE
_NCH = _CAP // _CS


@functools.partial(
    pl.kernel,
    out_type=(
        jax.ShapeDtypeStruct((_NW * _CAP,), jnp.int32),
        jax.ShapeDtypeStruct((_NW * _CAP,), jnp.int32),
        jax.ShapeDtypeStruct((_NW * _L,), jnp.int32),
    ),
    mesh=_sc_mesh,
    scratch_types=[
        pltpu.VMEM((_RANGE,), jnp.int32),
        pltpu.VMEM((_IDC,), jnp.int32),
        pltpu.VMEM((_CAP,), jnp.int32),
        pltpu.VMEM((_CAP,), jnp.int32),
        pltpu.VMEM((_L,), jnp.int32),
        pltpu.SemaphoreType.DMA,
    ],
    compiler_params=pltpu.CompilerParams(needs_layout_passes=False),
    cost_estimate=pl.CostEstimate(
        flops=0, transcendentals=0, bytes_accessed=200_000_000),
    name="sc_scatter_winner",
)
def _sc_winner(ids_hbm, srcs_hbm, tgts_hbm, cnts_hbm, winner, idsv,
               srcs, tgts, cnts_v, sem):
    wid = _wid()
    lanes = lax.iota(jnp.int32, _L)
    neg1 = jnp.full((_L,), -1, jnp.int32)

    @pl.loop(0, _RANGE // _L, unroll=8)
    def _init(j):
        winner[pl.ds(j * _L, _L)] = neg1

    for c in range(_NIDC):
        pltpu.sync_copy(ids_hbm.at[pl.ds(c * _IDC, _IDC)], idsv)

        @pl.loop(0, _IDC // _L, unroll=8)
        def _scan(j):
            vid = idsv[pl.ds(j * _L, _L)]
            inr = (vid >> _SHIFT) == wid
            slot = vid & (_RANGE - 1)
            pos = (c * _IDC + j * _L) + lanes
            _, last_m = plsc.scan_count(vid, inr)
            plsc.store_scatter(winner, [slot], pos, mask=last_m & inr)

    def _compact(j, off):
        w = winner[pl.ds(j * _L, _L)]
        m = w >= 0
        t = (wid * _RANGE + j * _L) + lanes
        plsc.store_compressed(srcs.at[pl.ds(off, _L)], w, mask=m)
        plsc.store_compressed(tgts.at[pl.ds(off, _L)], t, mask=m)
        return off + jnp.max(plsc.all_reduce_population_count(m))

    cnt = lax.fori_loop(0, _RANGE // _L, _compact, jnp.int32(0))

    cnt_pad = ((cnt + _CS - 1) // _CS) * _CS
    j0 = (cnt // _L) * _L

    @pl.when(cnt > 0)
    def _():
        sv = srcs[pl.ds(0, _L)]
        tv = tgts[pl.ds(0, _L)]
        vm = lanes < jnp.minimum(cnt, _L)
        t0s = jnp.max(jnp.where(vm, tv, -1))
        s0s = jnp.max(jnp.where(tv == t0s, sv, -1))
        s0 = jnp.full((_L,), s0s, jnp.int32)
        t0 = jnp.full((_L,), t0s, jnp.int32)

        @pl.loop(0, _CS // _L)
        def _pad(i):
            p0 = j0 + i * _L

            @pl.when(p0 < cnt_pad)
            def _():
                pm = (p0 + lanes) >= cnt
                srcs[pl.ds(p0, _L)] = jnp.where(pm, s0, srcs[pl.ds(p0, _L)])
                tgts[pl.ds(p0, _L)] = jnp.where(pm, t0, tgts[pl.ds(p0, _L)])

    cnts_v[...] = jnp.full((_L,), cnt, jnp.int32)
    pltpu.sync_copy(srcs, srcs_hbm.at[pl.ds(wid * _CAP, _CAP)])
    pltpu.sync_copy(tgts, tgts_hbm.at[pl.ds(wid * _CAP, _CAP)])
    pltpu.sync_copy(cnts_v, cnts_hbm.at[pl.ds(wid * _L, _L)])


@functools.partial(
    pl.kernel,
    out_type=(),
    mesh=_sc_mesh,
    scratch_types=[
        pltpu.VMEM((_CS,), jnp.int32),
        pltpu.VMEM((_CS,), jnp.int32),
        pltpu.VMEM((_CS,), jnp.int32),
        pltpu.VMEM((_CS,), jnp.int32),
        pltpu.VMEM((_CS, _MEM), jnp.float32),
        pltpu.VMEM((_CS, _MEM), jnp.float32),
        pltpu.VMEM((_L,), jnp.int32),
        pltpu.SemaphoreType.DMA,
        pltpu.SemaphoreType.DMA,
    ],
    compiler_params=pltpu.CompilerParams(needs_layout_passes=False),
    name="sc_scatter_move",
)
def _sc_move(srcs_hbm, tgts_hbm, cnts_hbm, newm_hbm, mem_ref,
             src_a, src_b, tgt_a, tgt_b, rows_a, rows_b, cnts_v, gsem, ssem):
    wid = _wid()
    base = wid * _CAP
    pltpu.sync_copy(cnts_hbm.at[pl.ds(wid * _L, _L)], cnts_v)
    cnt = jnp.max(cnts_v[...])
    nfull = cnt // _CS
    rem = cnt - nfull * _CS
    ntot = nfull + jnp.where(rem > 0, 1, 0)
    tail_off = ((jnp.maximum(cnt - _CS, 0) + 7) // 8) * 8
    src_bufs = [src_a, src_b]
    tgt_bufs = [tgt_a, tgt_b]
    row_bufs = [rows_a, rows_b]

    @pl.loop(0, _NCH, step=2)
    def _move(k):
        conds = [k + b < ntot for b in range(2)]
        offs = [base + jnp.where(k + b < nfull, (k + b) * _CS, tail_off)
                for b in range(2)]
        for b in range(2):
            @pl.when(conds[b])
            def _(b=b):
                off = offs[b]
                pltpu.sync_copy(srcs_hbm.at[pl.ds(off, _CS)], src_bufs[b])
                pltpu.sync_copy(tgts_hbm.at[pl.ds(off, _CS)], tgt_bufs[b])
                pltpu.async_copy(newm_hbm.at[src_bufs[b]], row_bufs[b], gsem)
        for b in range(2):
            @pl.when(conds[b])
            def _(b=b):
                pltpu.make_async_copy(
                    newm_hbm.at[src_bufs[b]], row_bufs[b], gsem).wait()
                pltpu.async_copy(row_bufs[b], mem_ref.at[tgt_bufs[b]], ssem)
        for b in range(2):
            @pl.when(conds[b])
            def _(b=b):
                pltpu.make_async_copy(
                    row_bufs[b], mem_ref.at[tgt_bufs[b]], ssem).wait()


def kernel(node_embeddings, node_ids, memory, W_ih, W_hh, b_ih, b_hh, W_cls, b_cls):
    ids_pad = jnp.concatenate(
        [node_ids, jnp.zeros((_B_PAD - _N,), jnp.int32)]
    )
    prev_pad = _sc_gather(memory, ids_pad)

    srcs, tgts, cnts = _sc_winner(node_ids)

    bf16 = jnp.bfloat16
    new_mem, logits_t = _tc_call(
        node_embeddings, prev_pad,
        W_ih.T.astype(bf16), W_hh.T.astype(bf16),
        b_ih[None, :], b_hh[None, :],
        W_cls[:, :_EMB].T.astype(bf16), W_cls[:, _EMB:].T.astype(bf16),
        b_cls[:, None],
    )

    memory_b = lax.optimization_barrier((memory, ids_pad))[0]
    mem_ref = jax.new_ref(memory_b)
    _sc_move(srcs, tgts, cnts, new_mem, mem_ref)
    return logits_t[:, :_N].T, mem_ref[...]


# speedup vs baseline: 1.0076x; 1.0076x over previous
import functools

import jax
import jax.numpy as jnp
from jax import lax
from jax.experimental import pallas as pl
from jax.experimental.pallas import tpu as pltpu
from jax.experimental.pallas import tpu_sc as plsc

_NC = 2
_NS = 16
_NW = _NC * _NS
_L = 16

_N = 50000
_EMB = 384
_MEM = 128
_MAX_NODES = 250000
_OUT = 2

_sc_mesh = plsc.VectorSubcoreMesh(
    core_axis_name="c", subcore_axis_name="s", num_cores=_NC, num_subcores=_NS
)


def _wid():
    return lax.axis_index("s") * _NC + lax.axis_index("c")


_BPW = 1568
_B_PAD = _BPW * _NW
_GCH = 224
_NGC = _BPW // _GCH


@functools.partial(
    pl.kernel,
    out_type=jax.ShapeDtypeStruct((_B_PAD, _MEM), jnp.float32),
    mesh=_sc_mesh,
    scratch_types=[
        pltpu.VMEM((_BPW,), jnp.int32),
        pltpu.VMEM((_GCH, _MEM), jnp.float32),
        pltpu.VMEM((_GCH, _MEM), jnp.float32),
        pltpu.VMEM((_GCH, _MEM), jnp.float32),
        pltpu.SemaphoreType.DMA,
        pltpu.SemaphoreType.DMA,
    ],
    name="sc_gather_rows",
)
def _sc_gather(mem_hbm, ids_hbm, out_hbm, idx_v, rows_a, rows_b, rows_c,
               gsem, ssem):
    base = _wid() * _BPW
    pltpu.sync_copy(ids_hbm.at[pl.ds(base, _BPW)], idx_v)
    bufs = [rows_a, rows_b, rows_c]

    def _gather(k):
        idx = idx_v.at[pl.ds(k * _GCH, _GCH)]
        return mem_hbm.at[idx], bufs[k % 3]

    def _scatter(k):
        return bufs[k % 3], out_hbm.at[pl.ds(base + k * _GCH, _GCH)]

    for k in range(min(3, _NGC)):
        pltpu.async_copy(*_gather(k), gsem)
    for k in range(_NGC):
        pltpu.make_async_copy(*_gather(k), gsem).wait()
        pltpu.async_copy(*_scatter(k), ssem)
        if k + 3 < _NGC:
            pltpu.make_async_copy(*_scatter(k), ssem).wait()
            pltpu.async_copy(*_gather(k + 3), gsem)
    for k in range(max(_NGC - 3, 0), _NGC):
        pltpu.make_async_copy(*_scatter(k), ssem).wait()


_RB = 1024


def _tc_body(emb_ref, prev_ref, wih_ref, whh_ref, bih_ref, bhh_ref,
             wce_ref, wcm_ref, bcls_ref, nm_ref, lg_ref):
    emb = emb_ref[...]
    prev = prev_ref[...]
    emb16 = emb.astype(jnp.bfloat16)
    prev16 = prev.astype(jnp.bfloat16)
    gi = jnp.dot(emb16, wih_ref[...], preferred_element_type=jnp.float32) + bih_ref[...]
    gh = jnp.dot(prev16, whh_ref[...], preferred_element_type=jnp.float32) + bhh_ref[...]
    r = jax.nn.sigmoid(gi[:, :_MEM] + gh[:, :_MEM])
    z = jax.nn.sigmoid(gi[:, _MEM:2 * _MEM] + gh[:, _MEM:2 * _MEM])
    n = jnp.tanh(gi[:, 2 * _MEM:] + r * gh[:, 2 * _MEM:])
    nm = (1.0 - z) * n + z * prev
    nm_ref[...] = nm
    dn = (((0,), (1,)), ((), ()))
    lg_ref[...] = (
        lax.dot_general(wce_ref[...], emb16, dn, preferred_element_type=jnp.float32)
        + lax.dot_general(wcm_ref[...], nm.astype(jnp.bfloat16), dn,
                          preferred_element_type=jnp.float32)
        + bcls_ref[...]
    )


_tc_call = pl.pallas_call(
    _tc_body,
    grid=(_B_PAD // _RB,),
    in_specs=[
        pl.BlockSpec((_RB, _EMB), lambda i: (i, 0)),
        pl.BlockSpec((_RB, _MEM), lambda i: (i, 0)),
        pl.BlockSpec((_EMB, 3 * _MEM), lambda i: (0, 0)),
        pl.BlockSpec((_MEM, 3 * _MEM), lambda i: (0, 0)),
        pl.BlockSpec((1, 3 * _MEM), lambda i: (0, 0)),
        pl.BlockSpec((1, 3 * _MEM), lambda i: (0, 0)),
        pl.BlockSpec((_EMB, _OUT), lambda i: (0, 0)),
        pl.BlockSpec((_MEM, _OUT), lambda i: (0, 0)),
        pl.BlockSpec((_OUT, 1), lambda i: (0, 0)),
    ],
    out_specs=[
        pl.BlockSpec((_RB, _MEM), lambda i: (i, 0)),
        pl.BlockSpec((_OUT, _RB), lambda i: (0, i)),
    ],
    out_shape=[
        jax.ShapeDtypeStruct((_B_PAD, _MEM), jnp.float32),
        jax.ShapeDtypeStruct((_OUT, _B_PAD), jnp.float32),
    ],
)


_RANGE = 8192
_SHIFT = 13
_IDC = 10000
_NIDC = _N // _IDC
_CS = 256
_CAP = _RANGE
_NCH = _CAP // _CS


@functools.partial(
    pl.kernel,
    out_type=(
        jax.ShapeDtypeStruct((_NW * _CAP,), jnp.int32),
        jax.ShapeDtypeStruct((_NW * _CAP,), jnp.int32),
        jax.ShapeDtypeStruct((_NW * _L,), jnp.int32),
    ),
    mesh=_sc_mesh,
    scratch_types=[
        pltpu.VMEM((_RANGE,), jnp.int32),
        pltpu.VMEM((_IDC,), jnp.int32),
        pltpu.VMEM((_CAP,), jnp.int32),
        pltpu.VMEM((_CAP,), jnp.int32),
        pltpu.VMEM((_L,), jnp.int32),
        pltpu.SemaphoreType.DMA,
    ],
    compiler_params=pltpu.CompilerParams(needs_layout_passes=False),
    cost_estimate=pl.CostEstimate(
        flops=0, transcendentals=0, bytes_accessed=200_000_000),
    name="sc_scatter_winner",
)
def _sc_winner(ids_hbm, srcs_hbm, tgts_hbm, cnts_hbm, winner, idsv,
               srcs, tgts, cnts_v, sem):
    wid = _wid()
    lanes = lax.iota(jnp.int32, _L)
    neg1 = jnp.full((_L,), -1, jnp.int32)

    @pl.loop(0, _RANGE // _L, unroll=8)
    def _init(j):
        winner[pl.ds(j * _L, _L)] = neg1

    for c in range(_NIDC):
        pltpu.sync_copy(ids_hbm.at[pl.ds(c * _IDC, _IDC)], idsv)

        @pl.loop(0, _IDC // _L, unroll=8)
        def _scan(j):
            vid = idsv[pl.ds(j * _L, _L)]
            inr = (vid >> _SHIFT) == wid
            slot = vid & (_RANGE - 1)
            pos = (c * _IDC + j * _L) + lanes
            _, last_m = plsc.scan_count(vid, inr)
            plsc.store_scatter(winner, [slot], pos, mask=last_m & inr)

    def _compact(j, off):
        w = winner[pl.ds(j * _L, _L)]
        m = w >= 0
        t = (wid * _RANGE + j * _L) + lanes
        plsc.store_compressed(srcs.at[pl.ds(off, _L)], w, mask=m)
        plsc.store_compressed(tgts.at[pl.ds(off, _L)], t, mask=m)
        return off + jnp.max(plsc.all_reduce_population_count(m))

    cnt = lax.fori_loop(0, _RANGE // _L, _compact, jnp.int32(0))

    cnt_pad = ((cnt + _CS - 1) // _CS) * _CS
    j0 = (cnt // _L) * _L

    @pl.when(cnt > 0)
    def _():
        sv = srcs[pl.ds(0, _L)]
        tv = tgts[pl.ds(0, _L)]
        vm = lanes < jnp.minimum(cnt, _L)
        t0s = jnp.max(jnp.where(vm, tv, -1))
        s0s = jnp.max(jnp.where(tv == t0s, sv, -1))
        s0 = jnp.full((_L,), s0s, jnp.int32)
        t0 = jnp.full((_L,), t0s, jnp.int32)

        @pl.loop(0, _CS // _L)
        def _pad(i):
            p0 = j0 + i * _L

            @pl.when(p0 < cnt_pad)
            def _():
                pm = (p0 + lanes) >= cnt
                srcs[pl.ds(p0, _L)] = jnp.where(pm, s0, srcs[pl.ds(p0, _L)])
                tgts[pl.ds(p0, _L)] = jnp.where(pm, t0, tgts[pl.ds(p0, _L)])

    cnts_v[...] = jnp.full((_L,), cnt, jnp.int32)
    pltpu.sync_copy(srcs, srcs_hbm.at[pl.ds(wid * _CAP, _CAP)])
    pltpu.sync_copy(tgts, tgts_hbm.at[pl.ds(wid * _CAP, _CAP)])
    pltpu.sync_copy(cnts_v, cnts_hbm.at[pl.ds(wid * _L, _L)])


@functools.partial(
    pl.kernel,
    out_type=(),
    mesh=_sc_mesh,
    scratch_types=[
        pltpu.VMEM((_CAP,), jnp.int32),
        pltpu.VMEM((_CAP,), jnp.int32),
        pltpu.VMEM((_CS,), jnp.int32),
        pltpu.VMEM((_CS,), jnp.int32),
        pltpu.VMEM((_CS,), jnp.int32),
        pltpu.VMEM((_CS, _MEM), jnp.float32),
        pltpu.VMEM((_CS, _MEM), jnp.float32),
        pltpu.VMEM((_CS, _MEM), jnp.float32),
        pltpu.VMEM((_L,), jnp.int32),
        pltpu.SemaphoreType.DMA,
        pltpu.SemaphoreType.DMA,
    ],
    compiler_params=pltpu.CompilerParams(needs_layout_passes=False),
    name="sc_scatter_move",
)
def _sc_move(srcs_hbm, tgts_hbm, cnts_hbm, newm_hbm, mem_ref,
             srcs_v, tgts_v, tgt_a, tgt_b, tgt_c,
             rows_a, rows_b, rows_c, cnts_v, gsem, ssem):
    wid = _wid()
    base = wid * _CAP
    pltpu.sync_copy(cnts_hbm.at[pl.ds(wid * _L, _L)], cnts_v)
    cnt = jnp.max(cnts_v[...])
    pltpu.sync_copy(srcs_hbm.at[pl.ds(base, _CAP)], srcs_v)
    pltpu.sync_copy(tgts_hbm.at[pl.ds(base, _CAP)], tgts_v)
    nfull = cnt // _CS
    rem = cnt - nfull * _CS
    ntot = nfull + jnp.where(rem > 0, 1, 0)
    tail_off = ((jnp.maximum(cnt - _CS, 0) + 7) // 8) * 8
    tgt_bufs = [tgt_a, tgt_b, tgt_c]
    row_bufs = [rows_a, rows_b, rows_c]

    def _off(k):
        return jnp.where(k < nfull, k * _CS, tail_off)

    @pl.loop(0, _NCH, step=3)
    def _move(k):
        conds = [k + b < ntot for b in range(3)]
        for b in range(3):
            @pl.when(conds[b])
            def _(b=b):
                off = _off(k + b)
                for j in range(_CS // _L):
                    tgt_bufs[b][pl.ds(j * _L, _L)] = (
                        tgts_v[pl.ds(off + j * _L, _L)])
                pltpu.async_copy(
                    newm_hbm.at[srcs_v.at[pl.ds(off, _CS)]], row_bufs[b], gsem)
        for b in range(3):
            @pl.when(conds[b])
            def _(b=b):
                off = _off(k + b)
                pltpu.make_async_copy(
                    newm_hbm.at[srcs_v.at[pl.ds(off, _CS)]], row_bufs[b],
                    gsem).wait()
                pltpu.async_copy(row_bufs[b], mem_ref.at[tgt_bufs[b]], ssem)
        for b in range(3):
            @pl.when(conds[b])
            def _(b=b):
                pltpu.make_async_copy(
                    row_bufs[b], mem_ref.at[tgt_bufs[b]], ssem).wait()


def kernel(node_embeddings, node_ids, memory, W_ih, W_hh, b_ih, b_hh, W_cls, b_cls):
    ids_pad = jnp.concatenate(
        [node_ids, jnp.zeros((_B_PAD - _N,), jnp.int32)]
    )
    prev_pad = _sc_gather(memory, ids_pad)

    srcs, tgts, cnts = _sc_winner(node_ids)

    bf16 = jnp.bfloat16
    new_mem, logits_t = _tc_call(
        node_embeddings, prev_pad,
        W_ih.T.astype(bf16), W_hh.T.astype(bf16),
        b_ih[None, :], b_hh[None, :],
        W_cls[:, :_EMB].T.astype(bf16), W_cls[:, _EMB:].T.astype(bf16),
        b_cls[:, None],
    )

    memory_b = lax.optimization_barrier((memory, ids_pad))[0]
    mem_ref = jax.new_ref(memory_b)
    _sc_move(srcs, tgts, cnts, new_mem, mem_ref)
    return logits_t[:, :_N].T, mem_ref[...]


# speedup vs baseline: 1.0761x; 1.0680x over previous
import functools

import jax
import jax.numpy as jnp
from jax import lax
from jax.experimental import pallas as pl
from jax.experimental.pallas import tpu as pltpu
from jax.experimental.pallas import tpu_sc as plsc

_NC = 2
_NS = 16
_NW = _NC * _NS
_L = 16

_N = 50000
_EMB = 384
_MEM = 128
_MAX_NODES = 250000
_OUT = 2

_sc_mesh = plsc.VectorSubcoreMesh(
    core_axis_name="c", subcore_axis_name="s", num_cores=_NC, num_subcores=_NS
)


def _wid():
    return lax.axis_index("s") * _NC + lax.axis_index("c")


_BPW = 1568
_B_PAD = _BPW * _NW
_GCH = 224
_NGC = _BPW // _GCH


@functools.partial(
    pl.kernel,
    out_type=jax.ShapeDtypeStruct((_B_PAD, _MEM), jnp.float32),
    mesh=_sc_mesh,
    scratch_types=[
        pltpu.VMEM((_BPW,), jnp.int32),
        pltpu.VMEM((_GCH, _MEM), jnp.float32),
        pltpu.VMEM((_GCH, _MEM), jnp.float32),
        pltpu.VMEM((_GCH, _MEM), jnp.float32),
        pltpu.SemaphoreType.DMA,
        pltpu.SemaphoreType.DMA,
    ],
    name="sc_gather_rows",
)
def _sc_gather(mem_hbm, ids_hbm, out_hbm, idx_v, rows_a, rows_b, rows_c,
               gsem, ssem):
    base = _wid() * _BPW
    pltpu.sync_copy(ids_hbm.at[pl.ds(base, _BPW)], idx_v)
    bufs = [rows_a, rows_b, rows_c]

    def _gather(k):
        idx = idx_v.at[pl.ds(k * _GCH, _GCH)]
        return mem_hbm.at[idx], bufs[k % 3]

    def _scatter(k):
        return bufs[k % 3], out_hbm.at[pl.ds(base + k * _GCH, _GCH)]

    for k in range(min(3, _NGC)):
        pltpu.async_copy(*_gather(k), gsem)
    for k in range(_NGC):
        pltpu.make_async_copy(*_gather(k), gsem).wait()
        pltpu.async_copy(*_scatter(k), ssem)
        if k + 3 < _NGC:
            pltpu.make_async_copy(*_scatter(k), ssem).wait()
            pltpu.async_copy(*_gather(k + 3), gsem)
    for k in range(max(_NGC - 3, 0), _NGC):
        pltpu.make_async_copy(*_scatter(k), ssem).wait()


_RB = 1024


_BKB = 5104


def _tc_body(emb_ref, prev_ref, wih_ref, whh_ref, bih_ref, bhh_ref,
             wce_ref, wcm_ref, bcls_ref, bank_ref, nm_ref, lg_ref, bko_ref):
    bko_ref[...] = bank_ref[...]
    emb = emb_ref[...]
    prev = prev_ref[...]
    emb16 = emb.astype(jnp.bfloat16)
    prev16 = prev.astype(jnp.bfloat16)
    gi = jnp.dot(emb16, wih_ref[...], preferred_element_type=jnp.float32) + bih_ref[...]
    gh = jnp.dot(prev16, whh_ref[...], preferred_element_type=jnp.float32) + bhh_ref[...]
    r = jax.nn.sigmoid(gi[:, :_MEM] + gh[:, :_MEM])
    z = jax.nn.sigmoid(gi[:, _MEM:2 * _MEM] + gh[:, _MEM:2 * _MEM])
    n = jnp.tanh(gi[:, 2 * _MEM:] + r * gh[:, 2 * _MEM:])
    nm = (1.0 - z) * n + z * prev
    nm_ref[...] = nm
    dn = (((0,), (1,)), ((), ()))
    lg_ref[...] = (
        lax.dot_general(wce_ref[...], emb16, dn, preferred_element_type=jnp.float32)
        + lax.dot_general(wcm_ref[...], nm.astype(jnp.bfloat16), dn,
                          preferred_element_type=jnp.float32)
        + bcls_ref[...]
    )


_tc_call = pl.pallas_call(
    _tc_body,
    grid=(_B_PAD // _RB,),
    in_specs=[
        pl.BlockSpec((_RB, _EMB), lambda i: (i, 0)),
        pl.BlockSpec((_RB, _MEM), lambda i: (i, 0)),
        pl.BlockSpec((_EMB, 3 * _MEM), lambda i: (0, 0)),
        pl.BlockSpec((_MEM, 3 * _MEM), lambda i: (0, 0)),
        pl.BlockSpec((1, 3 * _MEM), lambda i: (0, 0)),
        pl.BlockSpec((1, 3 * _MEM), lambda i: (0, 0)),
        pl.BlockSpec((_EMB, _OUT), lambda i: (0, 0)),
        pl.BlockSpec((_MEM, _OUT), lambda i: (0, 0)),
        pl.BlockSpec((_OUT, 1), lambda i: (0, 0)),
        pl.BlockSpec((_BKB, _MEM), lambda i: (i, 0)),
    ],
    out_specs=[
        pl.BlockSpec((_RB, _MEM), lambda i: (i, 0)),
        pl.BlockSpec((_OUT, _RB), lambda i: (0, i)),
        pl.BlockSpec((_BKB, _MEM), lambda i: (i, 0)),
    ],
    out_shape=[
        jax.ShapeDtypeStruct((_B_PAD, _MEM), jnp.float32),
        jax.ShapeDtypeStruct((_OUT, _B_PAD), jnp.float32),
        jax.ShapeDtypeStruct((_MAX_NODES, _MEM), jnp.float32),
    ],
)


_RANGE = 8192
_SHIFT = 13
_IDC = 10000
_NIDC = _N // _IDC
_CS = 256
_CAP = _RANGE
_NCH = _CAP // _CS


@functools.partial(
    pl.kernel,
    out_type=(
        jax.ShapeDtypeStruct((_NW * _CAP,), jnp.int32),
        jax.ShapeDtypeStruct((_NW * _CAP,), jnp.int32),
        jax.ShapeDtypeStruct((_NW * _L,), jnp.int32),
    ),
    mesh=_sc_mesh,
    scratch_types=[
        pltpu.VMEM((_RANGE,), jnp.int32),
        pltpu.VMEM((_IDC,), jnp.int32),
        pltpu.VMEM((_CAP,), jnp.int32),
        pltpu.VMEM((_CAP,), jnp.int32),
        pltpu.VMEM((_L,), jnp.int32),
        pltpu.SemaphoreType.DMA,
    ],
    compiler_params=pltpu.CompilerParams(needs_layout_passes=False),
    cost_estimate=pl.CostEstimate(
        flops=0, transcendentals=0, bytes_accessed=200_000_000),
    name="sc_scatter_winner",
)
def _sc_winner(ids_hbm, srcs_hbm, tgts_hbm, cnts_hbm, winner, idsv,
               srcs, tgts, cnts_v, sem):
    wid = _wid()
    lanes = lax.iota(jnp.int32, _L)
    neg1 = jnp.full((_L,), -1, jnp.int32)

    @pl.loop(0, _RANGE // _L, unroll=8)
    def _init(j):
        winner[pl.ds(j * _L, _L)] = neg1

    for c in range(_NIDC):
        pltpu.sync_copy(ids_hbm.at[pl.ds(c * _IDC, _IDC)], idsv)

        @pl.loop(0, _IDC // _L, unroll=8)
        def _scan(j):
            vid = idsv[pl.ds(j * _L, _L)]
            inr = (vid >> _SHIFT) == wid
            slot = vid & (_RANGE - 1)
            pos = (c * _IDC + j * _L) + lanes
            _, last_m = plsc.scan_count(vid, inr)
            plsc.store_scatter(winner, [slot], pos, mask=last_m & inr)

    def _compact(j, off):
        w = winner[pl.ds(j * _L, _L)]
        m = w >= 0
        t = (wid * _RANGE + j * _L) + lanes
        plsc.store_compressed(srcs.at[pl.ds(off, _L)], w, mask=m)
        plsc.store_compressed(tgts.at[pl.ds(off, _L)], t, mask=m)
        return off + jnp.max(plsc.all_reduce_population_count(m))

    cnt = lax.fori_loop(0, _RANGE // _L, _compact, jnp.int32(0))

    cnt_pad = ((cnt + _CS - 1) // _CS) * _CS
    j0 = (cnt // _L) * _L

    @pl.when(cnt > 0)
    def _():
        sv = srcs[pl.ds(0, _L)]
        tv = tgts[pl.ds(0, _L)]
        vm = lanes < jnp.minimum(cnt, _L)
        t0s = jnp.max(jnp.where(vm, tv, -1))
        s0s = jnp.max(jnp.where(tv == t0s, sv, -1))
        s0 = jnp.full((_L,), s0s, jnp.int32)
        t0 = jnp.full((_L,), t0s, jnp.int32)

        @pl.loop(0, _CS // _L)
        def _pad(i):
            p0 = j0 + i * _L

            @pl.when(p0 < cnt_pad)
            def _():
                pm = (p0 + lanes) >= cnt
                srcs[pl.ds(p0, _L)] = jnp.where(pm, s0, srcs[pl.ds(p0, _L)])
                tgts[pl.ds(p0, _L)] = jnp.where(pm, t0, tgts[pl.ds(p0, _L)])

    cnts_v[...] = jnp.full((_L,), cnt, jnp.int32)
    pltpu.sync_copy(srcs, srcs_hbm.at[pl.ds(wid * _CAP, _CAP)])
    pltpu.sync_copy(tgts, tgts_hbm.at[pl.ds(wid * _CAP, _CAP)])
    pltpu.sync_copy(cnts_v, cnts_hbm.at[pl.ds(wid * _L, _L)])


@functools.partial(
    pl.kernel,
    out_type=(),
    mesh=_sc_mesh,
    scratch_types=[
        pltpu.VMEM((_CAP,), jnp.int32),
        pltpu.VMEM((_CAP,), jnp.int32),
        pltpu.VMEM((_CS,), jnp.int32),
        pltpu.VMEM((_CS,), jnp.int32),
        pltpu.VMEM((_CS,), jnp.int32),
        pltpu.VMEM((_CS, _MEM), jnp.float32),
        pltpu.VMEM((_CS, _MEM), jnp.float32),
        pltpu.VMEM((_CS, _MEM), jnp.float32),
        pltpu.VMEM((_L,), jnp.int32),
        pltpu.SemaphoreType.DMA,
        pltpu.SemaphoreType.DMA,
    ],
    compiler_params=pltpu.CompilerParams(needs_layout_passes=False),
    name="sc_scatter_move",
)
def _sc_move(srcs_hbm, tgts_hbm, cnts_hbm, newm_hbm, mem_ref,
             srcs_v, tgts_v, tgt_a, tgt_b, tgt_c,
             rows_a, rows_b, rows_c, cnts_v, gsem, ssem):
    wid = _wid()
    base = wid * _CAP
    pltpu.sync_copy(cnts_hbm.at[pl.ds(wid * _L, _L)], cnts_v)
    cnt = jnp.max(cnts_v[...])
    pltpu.sync_copy(srcs_hbm.at[pl.ds(base, _CAP)], srcs_v)
    pltpu.sync_copy(tgts_hbm.at[pl.ds(base, _CAP)], tgts_v)
    nfull = cnt // _CS
    rem = cnt - nfull * _CS
    ntot = nfull + jnp.where(rem > 0, 1, 0)
    tail_off = ((jnp.maximum(cnt - _CS, 0) + 7) // 8) * 8
    tgt_bufs = [tgt_a, tgt_b, tgt_c]
    row_bufs = [rows_a, rows_b, rows_c]

    def _off(k):
        return jnp.where(k < nfull, k * _CS, tail_off)

    @pl.loop(0, _NCH, step=3)
    def _move(k):
        conds = [k + b < ntot for b in range(3)]
        for b in range(3):
            @pl.when(conds[b])
            def _(b=b):
                off = _off(k + b)
                for j in range(_CS // _L):
                    tgt_bufs[b][pl.ds(j * _L, _L)] = (
                        tgts_v[pl.ds(off + j * _L, _L)])
                pltpu.async_copy(
                    newm_hbm.at[srcs_v.at[pl.ds(off, _CS)]], row_bufs[b], gsem)
        for b in range(3):
            @pl.when(conds[b])
            def _(b=b):
                off = _off(k + b)
                pltpu.make_async_copy(
                    newm_hbm.at[srcs_v.at[pl.ds(off, _CS)]], row_bufs[b],
                    gsem).wait()
                pltpu.async_copy(row_bufs[b], mem_ref.at[tgt_bufs[b]], ssem)
        for b in range(3):
            @pl.when(conds[b])
            def _(b=b):
                pltpu.make_async_copy(
                    row_bufs[b], mem_ref.at[tgt_bufs[b]], ssem).wait()


def kernel(node_embeddings, node_ids, memory, W_ih, W_hh, b_ih, b_hh, W_cls, b_cls):
    ids_pad = jnp.concatenate(
        [node_ids, jnp.zeros((_B_PAD - _N,), jnp.int32)]
    )
    prev_pad = _sc_gather(memory, ids_pad)

    srcs, tgts, cnts = _sc_winner(node_ids)

    bf16 = jnp.bfloat16
    new_mem, logits_t, bank_out = _tc_call(
        node_embeddings, prev_pad,
        W_ih.T.astype(bf16), W_hh.T.astype(bf16),
        b_ih[None, :], b_hh[None, :],
        W_cls[:, :_EMB].T.astype(bf16), W_cls[:, _EMB:].T.astype(bf16),
        b_cls[:, None],
        memory,
    )

    mem_ref = jax.new_ref(bank_out)
    _sc_move(srcs, tgts, cnts, new_mem, mem_ref)
    return logits_t[:, :_N].T, mem_ref[...]


# speedup vs baseline: 1.1136x; 1.0348x over previous
import functools

import jax
import jax.numpy as jnp
from jax import lax
from jax.experimental import pallas as pl
from jax.experimental.pallas import tpu as pltpu
from jax.experimental.pallas import tpu_sc as plsc

_NC = 2
_NS = 16
_NW = _NC * _NS
_L = 16

_N = 50000
_EMB = 384
_MEM = 128
_MAX_NODES = 250000
_OUT = 2

_sc_mesh = plsc.VectorSubcoreMesh(
    core_axis_name="c", subcore_axis_name="s", num_cores=_NC, num_subcores=_NS
)


def _wid():
    return lax.axis_index("s") * _NC + lax.axis_index("c")


_BPW = 1568
_B_PAD = _BPW * _NW
_GCH = 224
_NGC = _BPW // _GCH


@functools.partial(
    pl.kernel,
    out_type=jax.ShapeDtypeStruct((_B_PAD, _MEM), jnp.float32),
    mesh=_sc_mesh,
    scratch_types=[
        pltpu.VMEM((_BPW,), jnp.int32),
        pltpu.VMEM((_GCH, _MEM), jnp.float32),
        pltpu.VMEM((_GCH, _MEM), jnp.float32),
        pltpu.VMEM((_GCH, _MEM), jnp.float32),
        pltpu.SemaphoreType.DMA,
        pltpu.SemaphoreType.DMA,
    ],
    name="sc_gather_rows",
)
def _sc_gather(mem_hbm, ids_hbm, out_hbm, idx_v, rows_a, rows_b, rows_c,
               gsem, ssem):
    base = _wid() * _BPW
    pltpu.sync_copy(ids_hbm.at[pl.ds(base, _BPW)], idx_v)
    bufs = [rows_a, rows_b, rows_c]

    def _gather(k):
        idx = idx_v.at[pl.ds(k * _GCH, _GCH)]
        return mem_hbm.at[idx], bufs[k % 3]

    def _scatter(k):
        return bufs[k % 3], out_hbm.at[pl.ds(base + k * _GCH, _GCH)]

    for k in range(min(3, _NGC)):
        pltpu.async_copy(*_gather(k), gsem)
    for k in range(_NGC):
        pltpu.make_async_copy(*_gather(k), gsem).wait()
        pltpu.async_copy(*_scatter(k), ssem)
        if k + 3 < _NGC:
            pltpu.make_async_copy(*_scatter(k), ssem).wait()
            pltpu.async_copy(*_gather(k + 3), gsem)
    for k in range(max(_NGC - 3, 0), _NGC):
        pltpu.make_async_copy(*_scatter(k), ssem).wait()


_RB = 1024


_BKB = 5104


def _tc_body(emb_ref, prev_ref, wih_ref, whh_ref, bih_ref, bhh_ref,
             wce_ref, wcm_ref, bcls_ref, bank_ref, nm_ref, lg_ref, bko_ref):
    bko_ref[...] = bank_ref[...]
    emb = emb_ref[...]
    prev = prev_ref[...]
    emb16 = emb.astype(jnp.bfloat16)
    prev16 = prev.astype(jnp.bfloat16)
    gi = jnp.dot(emb16, wih_ref[...], preferred_element_type=jnp.float32) + bih_ref[...]
    gh = jnp.dot(prev16, whh_ref[...], preferred_element_type=jnp.float32) + bhh_ref[...]
    r = jax.nn.sigmoid(gi[:, :_MEM] + gh[:, :_MEM])
    z = jax.nn.sigmoid(gi[:, _MEM:2 * _MEM] + gh[:, _MEM:2 * _MEM])
    n = jnp.tanh(gi[:, 2 * _MEM:] + r * gh[:, 2 * _MEM:])
    nm = (1.0 - z) * n + z * prev
    nm_ref[...] = nm
    dn = (((0,), (1,)), ((), ()))
    lg_ref[...] = (
        lax.dot_general(wce_ref[...], emb16, dn, preferred_element_type=jnp.float32)
        + lax.dot_general(wcm_ref[...], nm.astype(jnp.bfloat16), dn,
                          preferred_element_type=jnp.float32)
        + bcls_ref[...]
    )


_tc_call = pl.pallas_call(
    _tc_body,
    grid=(_B_PAD // _RB,),
    in_specs=[
        pl.BlockSpec((_RB, _EMB), lambda i: (i, 0)),
        pl.BlockSpec((_RB, _MEM), lambda i: (i, 0)),
        pl.BlockSpec((_EMB, 3 * _MEM), lambda i: (0, 0)),
        pl.BlockSpec((_MEM, 3 * _MEM), lambda i: (0, 0)),
        pl.BlockSpec((1, 3 * _MEM), lambda i: (0, 0)),
        pl.BlockSpec((1, 3 * _MEM), lambda i: (0, 0)),
        pl.BlockSpec((_EMB, _OUT), lambda i: (0, 0)),
        pl.BlockSpec((_MEM, _OUT), lambda i: (0, 0)),
        pl.BlockSpec((_OUT, 1), lambda i: (0, 0)),
        pl.BlockSpec((_BKB, _MEM), lambda i: (i, 0)),
    ],
    out_specs=[
        pl.BlockSpec((_RB, _MEM), lambda i: (i, 0)),
        pl.BlockSpec((_OUT, _RB), lambda i: (0, i)),
        pl.BlockSpec((_BKB, _MEM), lambda i: (i, 0)),
    ],
    out_shape=[
        jax.ShapeDtypeStruct((_B_PAD, _MEM), jnp.float32),
        jax.ShapeDtypeStruct((_OUT, _B_PAD), jnp.float32),
        jax.ShapeDtypeStruct((_MAX_NODES, _MEM), jnp.float32),
    ],
)


_RANGE = 8192
_SHIFT = 13
_IDC = 10000
_NIDC = _N // _IDC
_CS = 256
_CAP = _RANGE
_NCH = _CAP // _CS


@functools.partial(
    pl.kernel,
    out_type=(
        jax.ShapeDtypeStruct((_NW * _CAP,), jnp.int32),
        jax.ShapeDtypeStruct((_NW * _CAP,), jnp.int32),
        jax.ShapeDtypeStruct((_NW * _L,), jnp.int32),
    ),
    mesh=_sc_mesh,
    scratch_types=[
        pltpu.VMEM((_RANGE,), jnp.int32),
        pltpu.VMEM((_IDC,), jnp.int32),
        pltpu.VMEM((_CAP,), jnp.int32),
        pltpu.VMEM((_CAP,), jnp.int32),
        pltpu.VMEM((_L,), jnp.int32),
        pltpu.SemaphoreType.DMA,
    ],
    compiler_params=pltpu.CompilerParams(needs_layout_passes=False),
    cost_estimate=pl.CostEstimate(
        flops=0, transcendentals=0, bytes_accessed=200_000_000),
    name="sc_scatter_winner",
)
def _sc_winner(ids_hbm, srcs_hbm, tgts_hbm, cnts_hbm, winner, idsv,
               srcs, tgts, cnts_v, sem):
    wid = _wid()
    lanes = lax.iota(jnp.int32, _L)
    neg1 = jnp.full((_L,), -1, jnp.int32)

    @pl.loop(0, _RANGE // _L, unroll=8)
    def _init(j):
        winner[pl.ds(j * _L, _L)] = neg1

    for c in range(_NIDC):
        pltpu.sync_copy(ids_hbm.at[pl.ds(c * _IDC, _IDC)], idsv)

        @pl.loop(0, _IDC // _L, unroll=8)
        def _scan(j):
            vid = idsv[pl.ds(j * _L, _L)]
            inr = (vid >> _SHIFT) == wid
            slot = vid & (_RANGE - 1)
            pos = (c * _IDC + j * _L) + lanes
            _, last_m = plsc.scan_count(vid, inr)
            plsc.store_scatter(winner, [slot], pos, mask=last_m & inr)

    def _compact(j, off):
        w = winner[pl.ds(j * _L, _L)]
        m = w >= 0
        t = (wid * _RANGE + j * _L) + lanes
        plsc.store_compressed(srcs.at[pl.ds(off, _L)], w, mask=m)
        plsc.store_compressed(tgts.at[pl.ds(off, _L)], t, mask=m)
        return off + jnp.max(plsc.all_reduce_population_count(m))

    cnt = lax.fori_loop(0, _RANGE // _L, _compact, jnp.int32(0))

    cnt_pad = ((cnt + _CS - 1) // _CS) * _CS
    j0 = (cnt // _L) * _L

    @pl.when(cnt > 0)
    def _():
        sv = srcs[pl.ds(0, _L)]
        tv = tgts[pl.ds(0, _L)]
        vm = lanes < jnp.minimum(cnt, _L)
        t0s = jnp.max(jnp.where(vm, tv, -1))
        s0s = jnp.max(jnp.where(tv == t0s, sv, -1))
        s0 = jnp.full((_L,), s0s, jnp.int32)
        t0 = jnp.full((_L,), t0s, jnp.int32)

        @pl.loop(0, _CS // _L)
        def _pad(i):
            p0 = j0 + i * _L

            @pl.when(p0 < cnt_pad)
            def _():
                pm = (p0 + lanes) >= cnt
                srcs[pl.ds(p0, _L)] = jnp.where(pm, s0, srcs[pl.ds(p0, _L)])
                tgts[pl.ds(p0, _L)] = jnp.where(pm, t0, tgts[pl.ds(p0, _L)])

    cnts_v[...] = jnp.full((_L,), cnt, jnp.int32)
    pltpu.sync_copy(srcs, srcs_hbm.at[pl.ds(wid * _CAP, _CAP)])
    pltpu.sync_copy(tgts, tgts_hbm.at[pl.ds(wid * _CAP, _CAP)])
    pltpu.sync_copy(cnts_v, cnts_hbm.at[pl.ds(wid * _L, _L)])


@functools.partial(
    pl.kernel,
    out_type=(),
    mesh=_sc_mesh,
    scratch_types=[
        pltpu.VMEM((_CAP,), jnp.int32),
        pltpu.VMEM((_CAP,), jnp.int32),
        pltpu.VMEM((_CS,), jnp.int32),
        pltpu.VMEM((_CS,), jnp.int32),
        pltpu.VMEM((_CS,), jnp.int32),
        pltpu.VMEM((_CS, _MEM), jnp.float32),
        pltpu.VMEM((_CS, _MEM), jnp.float32),
        pltpu.VMEM((_CS, _MEM), jnp.float32),
        pltpu.VMEM((_L,), jnp.int32),
        pltpu.SemaphoreType.DMA,
        pltpu.SemaphoreType.DMA,
    ],
    compiler_params=pltpu.CompilerParams(needs_layout_passes=False),
    name="sc_scatter_move",
)
def _sc_move(srcs_hbm, tgts_hbm, cnts_hbm, newm_hbm, mem_ref,
             srcs_v, tgts_v, tgt_a, tgt_b, tgt_c,
             rows_a, rows_b, rows_c, cnts_v, gsem, ssem):
    wid = _wid()
    base = wid * _CAP
    pltpu.sync_copy(cnts_hbm.at[pl.ds(wid * _L, _L)], cnts_v)
    cnt = jnp.max(cnts_v[...])
    pltpu.sync_copy(srcs_hbm.at[pl.ds(base, _CAP)], srcs_v)
    pltpu.sync_copy(tgts_hbm.at[pl.ds(base, _CAP)], tgts_v)
    nfull = cnt // _CS
    rem = cnt - nfull * _CS
    ntot = nfull + jnp.where(rem > 0, 1, 0)
    tail_off = ((jnp.maximum(cnt - _CS, 0) + 7) // 8) * 8
    tgt_bufs = [tgt_a, tgt_b, tgt_c]
    row_bufs = [rows_a, rows_b, rows_c]

    def _off(k):
        return jnp.where(k < nfull, k * _CS, tail_off)

    @pl.loop(0, _NCH, step=3)
    def _move(k):
        conds = [k + b < ntot for b in range(3)]
        for b in range(3):
            @pl.when(conds[b])
            def _(b=b):
                off = _off(k + b)
                for j in range(_CS // _L):
                    tgt_bufs[b][pl.ds(j * _L, _L)] = (
                        tgts_v[pl.ds(off + j * _L, _L)])
                pltpu.async_copy(
                    newm_hbm.at[srcs_v.at[pl.ds(off, _CS)]], row_bufs[b], gsem)
        for b in range(3):
            @pl.when(conds[b])
            def _(b=b):
                off = _off(k + b)
                pltpu.make_async_copy(
                    newm_hbm.at[srcs_v.at[pl.ds(off, _CS)]], row_bufs[b],
                    gsem).wait()
                pltpu.async_copy(row_bufs[b], mem_ref.at[tgt_bufs[b]], ssem)
        for b in range(3):
            @pl.when(conds[b])
            def _(b=b):
                pltpu.make_async_copy(
                    row_bufs[b], mem_ref.at[tgt_bufs[b]], ssem).wait()


def kernel(node_embeddings, node_ids, memory, W_ih, W_hh, b_ih, b_hh, W_cls, b_cls):
    ids_pad = jnp.concatenate(
        [node_ids, jnp.arange(_B_PAD - _N, dtype=jnp.int32)]
    )
    prev_pad = _sc_gather(memory, ids_pad)

    srcs, tgts, cnts = _sc_winner(node_ids)

    bf16 = jnp.bfloat16
    new_mem, logits_t, bank_out = _tc_call(
        node_embeddings, prev_pad,
        W_ih.T.astype(bf16), W_hh.T.astype(bf16),
        b_ih[None, :], b_hh[None, :],
        W_cls[:, :_EMB].T.astype(bf16), W_cls[:, _EMB:].T.astype(bf16),
        b_cls[:, None],
        memory,
    )

    mem_ref = jax.new_ref(bank_out)
    _sc_move(srcs, tgts, cnts, new_mem, mem_ref)
    return logits_t[:, :_N].T, mem_ref[...]
